# baseline (speedup 1.0000x reference)
import functools

import jax
import jax.numpy as jnp
from jax import lax
from jax.experimental import pallas as pl
from jax.experimental.pallas import tpu as pltpu
from jax.experimental.pallas import tpu_sc as plsc

B = 16384
D = 32
H = 128
V = 1000000
NC = 2
NS = 16
NW = NC * NS
BPW = B // NW
CHUNK = BPW // 2
CB = 6144
NBLK = -(-V // CB)


def _packmm_body(x_ref, w_ref, o_ref):
    xb = x_ref[...].astype(jnp.bfloat16)
    wb = w_ref[...].astype(jnp.bfloat16)
    o_ref[...] = lax.dot_general(
        xb, wb, (((0,), (0,)), ((), ())),
        preferred_element_type=jnp.float32)


def _packmm_tc(table_t, w_half):
    return pl.pallas_call(
        _packmm_body,
        grid=(NBLK,),
        in_specs=[
            pl.BlockSpec((D, CB), lambda i: (0, i)),
            pl.BlockSpec((D, H), lambda i: (0, 0)),
        ],
        out_specs=pl.BlockSpec((CB, H), lambda i: (i, 0)),
        out_shape=jax.ShapeDtypeStruct((V, H), jnp.float32),
        compiler_params=pltpu.CompilerParams(
            dimension_semantics=("arbitrary",)),
    )(table_t, w_half)


def _gather_one(table, idx):
    mesh = plsc.VectorSubcoreMesh(core_axis_name="c", subcore_axis_name="s")

    @functools.partial(
        pl.kernel,
        mesh=mesh,
        out_type=jax.ShapeDtypeStruct((B, H), jnp.float32),
        scratch_types=[
            pltpu.VMEM((BPW,), jnp.int32),
            pltpu.VMEM((CHUNK, H), jnp.float32),
            pltpu.VMEM((CHUNK, H), jnp.float32),
            pltpu.SemaphoreType.DMA,
            pltpu.SemaphoreType.DMA,
            pltpu.SemaphoreType.DMA,
            pltpu.SemaphoreType.DMA,
        ],
    )
    def k(t_hbm, i_hbm, o_hbm, idx_v, buf0, buf1, gs0, gs1, ws0, ws1):
        wid = lax.axis_index("s") * NC + lax.axis_index("c")
        base = wid * BPW
        pltpu.sync_copy(i_hbm.at[pl.ds(base, BPW)], idx_v)
        g0 = pltpu.async_copy(t_hbm.at[idx_v.at[pl.ds(0, CHUNK)]], buf0, gs0)
        g1 = pltpu.async_copy(t_hbm.at[idx_v.at[pl.ds(CHUNK, CHUNK)]], buf1, gs1)
        g0.wait()
        w0 = pltpu.async_copy(buf0, o_hbm.at[pl.ds(base, CHUNK)], ws0)
        g1.wait()
        w1 = pltpu.async_copy(buf1, o_hbm.at[pl.ds(base + CHUNK, CHUNK)], ws1)
        w0.wait()
        w1.wait()

    return k(table, idx)


def _mlp_body(gu_ref, gi_ref, b1_ref, w2_ref, b2_ref, o_ref):
    h = jnp.maximum(gu_ref[...] + gi_ref[...] + b1_ref[...], 0.0)
    o_ref[...] = jnp.sum(h * w2_ref[...], axis=1, keepdims=True) + b2_ref[...]


def _mlp_tc(gu, gi, b1, W2, b2):
    b1r = b1.reshape(1, H)
    w2r = W2.reshape(1, H)
    b2s = b2.reshape(1, 1)

    BLK = 2048
    return pl.pallas_call(
        _mlp_body,
        grid=(B // BLK,),
        in_specs=[
            pl.BlockSpec((BLK, H), lambda i: (i, 0)),
            pl.BlockSpec((BLK, H), lambda i: (i, 0)),
            pl.BlockSpec((1, H), lambda i: (0, 0)),
            pl.BlockSpec((1, H), lambda i: (0, 0)),
            pl.BlockSpec((1, 1), lambda i: (0, 0)),
        ],
        out_specs=pl.BlockSpec((BLK, 1), lambda i: (i, 0)),
        out_shape=jax.ShapeDtypeStruct((B, 1), jnp.float32),
    )(gu, gi, b1r, w2r, b2s)


def kernel(user, item, user_table, item_table, W1, b1, W2, b2):
    tu = _packmm_tc(user_table.T, W1[:D])
    gu = _gather_one(tu, user)
    ti = _packmm_tc(item_table.T, W1[D:])
    gi = _gather_one(ti, item)
    return _mlp_tc(gu, gi, b1, W2, b2)


# speedup vs baseline: 1.1813x; 1.1813x over previous
import functools

import jax
import jax.numpy as jnp
from jax import lax
from jax.experimental import pallas as pl
from jax.experimental.pallas import tpu as pltpu
from jax.experimental.pallas import tpu_sc as plsc

B = 16384
D = 32
H = 128
V = 1000000
NC = 2
NS = 16
NW = NC * NS
BPW = B // NW
CHUNK = BPW // 2
CB = 12288
NBLK = -(-V // CB)


def _packmm_body(x_ref, w_ref, o_ref):
    xb = x_ref[...].astype(jnp.bfloat16)
    wb = w_ref[...].astype(jnp.bfloat16)
    o_ref[...] = lax.dot_general(
        xb, wb, (((0,), (0,)), ((), ())),
        preferred_element_type=jnp.float32)


def _packmm_tc(table_t, w_half):
    return pl.pallas_call(
        _packmm_body,
        grid=(NBLK,),
        in_specs=[
            pl.BlockSpec((D, CB), lambda i: (0, i)),
            pl.BlockSpec((D, H), lambda i: (0, 0)),
        ],
        out_specs=pl.BlockSpec((CB, H), lambda i: (i, 0)),
        out_shape=jax.ShapeDtypeStruct((V, H), jnp.float32),
        compiler_params=pltpu.CompilerParams(
            dimension_semantics=("arbitrary",)),
    )(table_t, w_half)


def _gather_one(table, idx):
    mesh = plsc.VectorSubcoreMesh(core_axis_name="c", subcore_axis_name="s")

    @functools.partial(
        pl.kernel,
        mesh=mesh,
        out_type=jax.ShapeDtypeStruct((B, H), jnp.float32),
        scratch_types=[
            pltpu.VMEM((BPW,), jnp.int32),
            pltpu.VMEM((CHUNK, H), jnp.float32),
            pltpu.VMEM((CHUNK, H), jnp.float32),
            pltpu.SemaphoreType.DMA,
            pltpu.SemaphoreType.DMA,
            pltpu.SemaphoreType.DMA,
            pltpu.SemaphoreType.DMA,
        ],
    )
    def k(t_hbm, i_hbm, o_hbm, idx_v, buf0, buf1, gs0, gs1, ws0, ws1):
        wid = lax.axis_index("s") * NC + lax.axis_index("c")
        base = wid * BPW
        pltpu.sync_copy(i_hbm.at[pl.ds(base, BPW)], idx_v)
        g0 = pltpu.async_copy(t_hbm.at[idx_v.at[pl.ds(0, CHUNK)]], buf0, gs0)
        g1 = pltpu.async_copy(t_hbm.at[idx_v.at[pl.ds(CHUNK, CHUNK)]], buf1, gs1)
        g0.wait()
        w0 = pltpu.async_copy(buf0, o_hbm.at[pl.ds(base, CHUNK)], ws0)
        g1.wait()
        w1 = pltpu.async_copy(buf1, o_hbm.at[pl.ds(base + CHUNK, CHUNK)], ws1)
        w0.wait()
        w1.wait()

    return k(table, idx)


def _mlp_body(gu_ref, gi_ref, b1_ref, w2_ref, b2_ref, o_ref):
    h = jnp.maximum(gu_ref[...] + gi_ref[...] + b1_ref[...], 0.0)
    o_ref[...] = jnp.sum(h * w2_ref[...], axis=1, keepdims=True) + b2_ref[...]


def _mlp_tc(gu, gi, b1, W2, b2):
    b1r = b1.reshape(1, H)
    w2r = W2.reshape(1, H)
    b2s = b2.reshape(1, 1)

    BLK = 2048
    return pl.pallas_call(
        _mlp_body,
        grid=(B // BLK,),
        in_specs=[
            pl.BlockSpec((BLK, H), lambda i: (i, 0)),
            pl.BlockSpec((BLK, H), lambda i: (i, 0)),
            pl.BlockSpec((1, H), lambda i: (0, 0)),
            pl.BlockSpec((1, H), lambda i: (0, 0)),
            pl.BlockSpec((1, 1), lambda i: (0, 0)),
        ],
        out_specs=pl.BlockSpec((BLK, 1), lambda i: (i, 0)),
        out_shape=jax.ShapeDtypeStruct((B, 1), jnp.float32),
    )(gu, gi, b1r, w2r, b2s)


def kernel(user, item, user_table, item_table, W1, b1, W2, b2):
    tu = _packmm_tc(user_table.T, W1[:D])
    gu = _gather_one(tu, user)
    ti = _packmm_tc(item_table.T, W1[D:])
    gi = _gather_one(ti, item)
    return _mlp_tc(gu, gi, b1, W2, b2)
